# Initial kernel scaffold; baseline (speedup 1.0000x reference)
import jax
import jax.numpy as jnp
from jax.experimental import pallas as pl


def kernel(x, edge_index, edge_weight, node_lock, W, b):
    raise NotImplementedError("write your pallas kernel here")


# speedup vs baseline: 4.4253x; 4.4253x over previous
import functools

import jax
import jax.numpy as jnp
from jax import lax
from jax.experimental import pallas as pl
from jax.experimental.pallas import tpu as pltpu
from jax.experimental.pallas import tpu_sc as plsc

N = 10000
E = 320000
F = 128
NC = 2
NS = 16
LANES = 16

EPW = E // (NC * NS)
CHUNK = 80
NCHUNK = EPW // CHUNK
NPAD = 10240
ROWS_PT = NPAD // NS
WB = 128


def _matmul_body(x_ref, wt_ref, o_ref):
    o_ref[...] = jnp.dot(x_ref[...], wt_ref[...],
                         preferred_element_type=jnp.float32)


def _combine_body(p_ref, b_ref, o_ref):
    o_ref[...] = p_ref[0] + p_ref[1] + b_ref[...][None, :]


def _scatter_body(xlin, src_h, dst_h, w_h, out_h,
                  acc, src_v, dst_v, w_v, rows_v, zero_v, sem):
    c = lax.axis_index("c")
    s = lax.axis_index("s")
    w = c * NS + s

    def _zero_rows(r, _):
        for j in range(F // LANES):
            zero_v[r, pl.ds(j * LANES, LANES)] = jnp.zeros(
                (LANES,), jnp.float32)
        return 0
    lax.fori_loop(0, WB, _zero_rows, 0)
    for t in range(ROWS_PT // WB):
        pltpu.sync_copy(zero_v, acc.at[pl.ds(s * ROWS_PT + t * WB, WB)])
    plsc.subcore_barrier()

    def _chunk(k, _):
        eb = w * EPW + k * CHUNK
        pltpu.sync_copy(src_h.at[pl.ds(eb, CHUNK)], src_v)
        pltpu.sync_copy(dst_h.at[pl.ds(eb, CHUNK)], dst_v)
        pltpu.sync_copy(w_h.at[pl.ds(eb, CHUNK)], w_v)
        pltpu.async_copy(xlin.at[src_v], rows_v, sem).wait()

        def _scale(g, _):
            base = g * LANES
            wvec = w_v[pl.ds(base, LANES)]
            for l in range(LANES):
                ew = wvec[l]
                for j in range(F // LANES):
                    sl = pl.ds(j * LANES, LANES)
                    rows_v[base + l, sl] = rows_v[base + l, sl] * ew
            return 0
        lax.fori_loop(0, CHUNK // LANES, _scale, 0)

        pltpu.sync_copy(rows_v, acc.at[dst_v], add=True)
        return 0
    lax.fori_loop(0, NCHUNK, _chunk, 0)
    plsc.subcore_barrier()

    for t in range(ROWS_PT // WB):
        r0 = s * ROWS_PT + t * WB
        pltpu.sync_copy(acc.at[pl.ds(r0, WB)], zero_v)
        pltpu.sync_copy(zero_v, out_h.at[c, pl.ds(r0, WB)])


_scatter_kernel = functools.partial(
    pl.kernel,
    out_type=jax.ShapeDtypeStruct((NC, NPAD, F), jnp.float32),
    mesh=plsc.VectorSubcoreMesh(core_axis_name="c", subcore_axis_name="s"),
    scratch_types=[
        pltpu.VMEM_SHARED((NPAD, F), jnp.float32),
        pltpu.VMEM((CHUNK,), jnp.int32),
        pltpu.VMEM((CHUNK,), jnp.int32),
        pltpu.VMEM((CHUNK,), jnp.float32),
        pltpu.VMEM((CHUNK, F), jnp.float32),
        pltpu.VMEM((WB, F), jnp.float32),
        pltpu.SemaphoreType.DMA,
    ],
)(_scatter_body)


@jax.jit
def kernel(x, edge_index, edge_weight, node_lock, W, b):
    x_lin = pl.pallas_call(
        _matmul_body,
        grid=(10,),
        in_specs=[
            pl.BlockSpec((N // 10, F), lambda i: (i, 0)),
            pl.BlockSpec((F, F), lambda i: (0, 0)),
        ],
        out_specs=pl.BlockSpec((N // 10, F), lambda i: (i, 0)),
        out_shape=jax.ShapeDtypeStruct((N, F), jnp.float32),
    )(x, W.T)

    partials = _scatter_kernel(x_lin, edge_index[0], edge_index[1],
                               edge_weight)

    out = pl.pallas_call(
        _combine_body,
        grid=(10,),
        in_specs=[
            pl.BlockSpec((NC, N // 10, F), lambda i: (0, i, 0)),
            pl.BlockSpec((F,), lambda i: (0,)),
        ],
        out_specs=pl.BlockSpec((N // 10, F), lambda i: (i, 0)),
        out_shape=jax.ShapeDtypeStruct((N, F), jnp.float32),
    )(partials, b)

    return (out, (edge_index, edge_weight))


# speedup vs baseline: 9.2409x; 2.0882x over previous
import functools

import jax
import jax.numpy as jnp
from jax import lax
from jax.experimental import pallas as pl
from jax.experimental.pallas import tpu as pltpu
from jax.experimental.pallas import tpu_sc as plsc

N = 10000
E = 320000
F = 128
NC = 2
NS = 16
LANES = 16
NW = NC * NS

CHUNK = 128
RPW = 80
EROWS = NW * RPW
EPAD = EROWS * CHUNK
BLK = 16
NBLK = RPW // BLK
NIT = BLK // 2

NPAD = 10240
ROWS_PT = NPAD // NS
WB = 32


def _matmul_body(x_ref, wt_ref, o_ref):
    o_ref[...] = jnp.dot(x_ref[...], wt_ref[...],
                         preferred_element_type=jnp.float32)


def _combine_body(p_ref, b_ref, o_ref):
    o_ref[...] = p_ref[0] + p_ref[1] + b_ref[...][None, :]


def _scatter_body(xlin, src_h, dst_h, w_h, out_h,
                  acc, src_all, dst_all, w_all, rows0, rows1, zero_v,
                  sg0, sg1, ss0, ss1):
    c = lax.axis_index("c")
    s = lax.axis_index("s")
    w = c * NS + s

    def _zero_rows(r, _):
        for j in range(F // LANES):
            zero_v[r, pl.ds(j * LANES, LANES)] = jnp.zeros(
                (LANES,), jnp.float32)
        return 0
    lax.fori_loop(0, WB, _zero_rows, 0)
    for t in range(ROWS_PT // WB):
        pltpu.sync_copy(zero_v, acc.at[pl.ds(s * ROWS_PT + t * WB, WB)])
    plsc.subcore_barrier()

    def _scale(rows_ref, k):
        def _grp(g, _):
            base = g * LANES
            wvec = w_all[k, pl.ds(base, LANES)]
            for l in range(LANES):
                ew = wvec[l]
                for j in range(F // LANES):
                    sl = pl.ds(j * LANES, LANES)
                    rows_ref[base + l, sl] = rows_ref[base + l, sl] * ew
            return 0
        lax.fori_loop(0, CHUNK // LANES, _grp, 0)

    row0 = w * RPW
    for blk in range(NBLK):
        brow = row0 + blk * BLK
        pltpu.sync_copy(src_h.at[pl.ds(brow, BLK)], src_all)
        pltpu.sync_copy(dst_h.at[pl.ds(brow, BLK)], dst_all)
        pltpu.sync_copy(w_h.at[pl.ds(brow, BLK)], w_all)

        pltpu.async_copy(xlin.at[src_all.at[0]], rows0, sg0)

        def _iter(k2, _):
            k = 2 * k2
            pltpu.make_async_copy(xlin.at[src_all.at[k]], rows0, sg0).wait()

            @pl.when(k2 > 0)
            def _():
                pltpu.make_async_copy(
                    rows1, acc.at[dst_all.at[k]], ss1).wait()
            pltpu.async_copy(xlin.at[src_all.at[k + 1]], rows1, sg1)

            _scale(rows0, k)
            pltpu.async_copy(rows0, acc.at[dst_all.at[k]], ss0, add=True)

            pltpu.make_async_copy(
                xlin.at[src_all.at[k + 1]], rows1, sg1).wait()
            _scale(rows1, k + 1)
            pltpu.async_copy(rows1, acc.at[dst_all.at[k + 1]], ss1, add=True)

            @pl.when(k2 < NIT - 1)
            def _():
                pltpu.make_async_copy(
                    rows0, acc.at[dst_all.at[k]], ss0).wait()
                pltpu.async_copy(xlin.at[src_all.at[k + 2]], rows0, sg0)
            return 0
        lax.fori_loop(0, NIT, _iter, 0)
        pltpu.make_async_copy(rows0, acc.at[dst_all.at[BLK - 2]], ss0).wait()
        pltpu.make_async_copy(rows1, acc.at[dst_all.at[BLK - 1]], ss1).wait()
    plsc.subcore_barrier()

    pltpu.sync_copy(acc.at[pl.ds(s * ROWS_PT, ROWS_PT)],
                    out_h.at[c, pl.ds(s * ROWS_PT, ROWS_PT)])


_scatter_kernel = functools.partial(
    pl.kernel,
    out_type=jax.ShapeDtypeStruct((NC, NPAD, F), jnp.float32),
    mesh=plsc.VectorSubcoreMesh(core_axis_name="c", subcore_axis_name="s"),
    scratch_types=[
        pltpu.VMEM_SHARED((NPAD, F), jnp.float32),
        pltpu.VMEM((BLK, CHUNK), jnp.int32),
        pltpu.VMEM((BLK, CHUNK), jnp.int32),
        pltpu.VMEM((BLK, CHUNK), jnp.float32),
        pltpu.VMEM((CHUNK, F), jnp.float32),
        pltpu.VMEM((CHUNK, F), jnp.float32),
        pltpu.VMEM((WB, F), jnp.float32),
        pltpu.SemaphoreType.DMA,
        pltpu.SemaphoreType.DMA,
        pltpu.SemaphoreType.DMA,
        pltpu.SemaphoreType.DMA,
    ],
)(_scatter_body)


@jax.jit
def kernel(x, edge_index, edge_weight, node_lock, W, b):
    x_lin = pl.pallas_call(
        _matmul_body,
        grid=(10,),
        in_specs=[
            pl.BlockSpec((N // 10, F), lambda i: (i, 0)),
            pl.BlockSpec((F, F), lambda i: (0, 0)),
        ],
        out_specs=pl.BlockSpec((N // 10, F), lambda i: (i, 0)),
        out_shape=jax.ShapeDtypeStruct((N, F), jnp.float32),
    )(x, W.T)

    npad_e = EPAD - E
    pad_idx = jnp.arange(npad_e, dtype=jnp.int32) % N
    srcp = jnp.concatenate([edge_index[0], pad_idx]).reshape(EROWS, CHUNK)
    dstp = jnp.concatenate([edge_index[1], pad_idx]).reshape(EROWS, CHUNK)
    wp = jnp.concatenate(
        [edge_weight, jnp.zeros((npad_e,), jnp.float32)]).reshape(EROWS, CHUNK)

    partials = _scatter_kernel(x_lin, srcp, dstp, wp)

    out = pl.pallas_call(
        _combine_body,
        grid=(10,),
        in_specs=[
            pl.BlockSpec((NC, N // 10, F), lambda i: (0, i, 0)),
            pl.BlockSpec((F,), lambda i: (0,)),
        ],
        out_specs=pl.BlockSpec((N // 10, F), lambda i: (i, 0)),
        out_shape=jax.ShapeDtypeStruct((N, F), jnp.float32),
    )(partials, b)

    return (out, (edge_index, edge_weight))


# speedup vs baseline: 9.2499x; 1.0010x over previous
import functools

import jax
import jax.numpy as jnp
from jax import lax
from jax.experimental import pallas as pl
from jax.experimental.pallas import tpu as pltpu
from jax.experimental.pallas import tpu_sc as plsc

N = 10000
E = 320000
F = 128
NC = 2
NS = 16
LANES = 16
NW = NC * NS

CHUNK = 128
RPW = 80
EROWS = NW * RPW
EPAD = EROWS * CHUNK
BLK = 16
NBLK = RPW // BLK
NIT = BLK // 2

NPAD = 10240
ROWS_PT = NPAD // NS
WB = 32


def _matmul_body(x_ref, wt_ref, o_ref):
    o_ref[...] = jnp.dot(x_ref[...], wt_ref[...],
                         preferred_element_type=jnp.float32)


def _combine_body(p_ref, b_ref, o_ref):
    o_ref[...] = p_ref[0] + p_ref[1] + b_ref[...][None, :]


def _scatter_body(xlin, src_h, dst_h, w_h, out_h,
                  acc, src_all, dst_all, w_all, rows0, rows1, zero_v,
                  sg0, sg1, ss0, ss1):
    c = lax.axis_index("c")
    s = lax.axis_index("s")
    w = c * NS + s

    def _zero_rows(r, _):
        for j in range(F // LANES):
            zero_v[r, pl.ds(j * LANES, LANES)] = jnp.zeros(
                (LANES,), jnp.float32)
        return 0
    lax.fori_loop(0, WB, _zero_rows, 0)
    for t in range(ROWS_PT // WB):
        pltpu.sync_copy(zero_v, acc.at[pl.ds(s * ROWS_PT + t * WB, WB)])
    plsc.subcore_barrier()

    def _scale(rows_ref, k):
        @plsc.parallel_loop(0, CHUNK // LANES, unroll=2)
        def _grp(g):
            base = g * LANES
            wvec = w_all[k, pl.ds(base, LANES)]
            for l in range(LANES):
                ew = wvec[l]
                for j in range(F // LANES):
                    sl = pl.ds(j * LANES, LANES)
                    rows_ref[base + l, sl] = rows_ref[base + l, sl] * ew

    row0 = w * RPW
    for blk in range(NBLK):
        brow = row0 + blk * BLK
        pltpu.sync_copy(src_h.at[pl.ds(brow, BLK)], src_all)
        pltpu.sync_copy(dst_h.at[pl.ds(brow, BLK)], dst_all)
        pltpu.sync_copy(w_h.at[pl.ds(brow, BLK)], w_all)

        pltpu.async_copy(xlin.at[src_all.at[0]], rows0, sg0)

        def _iter(k2, _):
            k = 2 * k2
            pltpu.make_async_copy(xlin.at[src_all.at[k]], rows0, sg0).wait()

            @pl.when(k2 > 0)
            def _():
                pltpu.make_async_copy(
                    rows1, acc.at[dst_all.at[k]], ss1).wait()
            pltpu.async_copy(xlin.at[src_all.at[k + 1]], rows1, sg1)

            _scale(rows0, k)
            pltpu.async_copy(rows0, acc.at[dst_all.at[k]], ss0, add=True)

            pltpu.make_async_copy(
                xlin.at[src_all.at[k + 1]], rows1, sg1).wait()
            _scale(rows1, k + 1)
            pltpu.async_copy(rows1, acc.at[dst_all.at[k + 1]], ss1, add=True)

            @pl.when(k2 < NIT - 1)
            def _():
                pltpu.make_async_copy(
                    rows0, acc.at[dst_all.at[k]], ss0).wait()
                pltpu.async_copy(xlin.at[src_all.at[k + 2]], rows0, sg0)
            return 0
        lax.fori_loop(0, NIT, _iter, 0)
        pltpu.make_async_copy(rows0, acc.at[dst_all.at[BLK - 2]], ss0).wait()
        pltpu.make_async_copy(rows1, acc.at[dst_all.at[BLK - 1]], ss1).wait()
    plsc.subcore_barrier()

    pltpu.sync_copy(acc.at[pl.ds(s * ROWS_PT, ROWS_PT)],
                    out_h.at[c, pl.ds(s * ROWS_PT, ROWS_PT)])


_scatter_kernel = functools.partial(
    pl.kernel,
    out_type=jax.ShapeDtypeStruct((NC, NPAD, F), jnp.float32),
    mesh=plsc.VectorSubcoreMesh(core_axis_name="c", subcore_axis_name="s"),
    scratch_types=[
        pltpu.VMEM_SHARED((NPAD, F), jnp.float32),
        pltpu.VMEM((BLK, CHUNK), jnp.int32),
        pltpu.VMEM((BLK, CHUNK), jnp.int32),
        pltpu.VMEM((BLK, CHUNK), jnp.float32),
        pltpu.VMEM((CHUNK, F), jnp.float32),
        pltpu.VMEM((CHUNK, F), jnp.float32),
        pltpu.VMEM((WB, F), jnp.float32),
        pltpu.SemaphoreType.DMA,
        pltpu.SemaphoreType.DMA,
        pltpu.SemaphoreType.DMA,
        pltpu.SemaphoreType.DMA,
    ],
)(_scatter_body)


@jax.jit
def kernel(x, edge_index, edge_weight, node_lock, W, b):
    x_lin = pl.pallas_call(
        _matmul_body,
        grid=(10,),
        in_specs=[
            pl.BlockSpec((N // 10, F), lambda i: (i, 0)),
            pl.BlockSpec((F, F), lambda i: (0, 0)),
        ],
        out_specs=pl.BlockSpec((N // 10, F), lambda i: (i, 0)),
        out_shape=jax.ShapeDtypeStruct((N, F), jnp.float32),
    )(x, W.T)

    npad_e = EPAD - E
    pad_idx = jnp.arange(npad_e, dtype=jnp.int32) % N
    srcp = jnp.concatenate([edge_index[0], pad_idx]).reshape(EROWS, CHUNK)
    dstp = jnp.concatenate([edge_index[1], pad_idx]).reshape(EROWS, CHUNK)
    wp = jnp.concatenate(
        [edge_weight, jnp.zeros((npad_e,), jnp.float32)]).reshape(EROWS, CHUNK)

    partials = _scatter_kernel(x_lin, srcp, dstp, wp)

    out = pl.pallas_call(
        _combine_body,
        grid=(10,),
        in_specs=[
            pl.BlockSpec((NC, N // 10, F), lambda i: (0, i, 0)),
            pl.BlockSpec((F,), lambda i: (0,)),
        ],
        out_specs=pl.BlockSpec((N // 10, F), lambda i: (i, 0)),
        out_shape=jax.ShapeDtypeStruct((N, F), jnp.float32),
    )(partials, b)

    return (out, (edge_index, edge_weight))
